# baseline (speedup 1.0000x reference)
import jax
import jax.numpy as jnp
from jax import lax
from jax.experimental import pallas as pl
from jax.experimental.pallas import tpu as pltpu
from jax.experimental.pallas import tpu_sc as plsc

N = 10000
D = 128
N_PAD = 10240
TRASH = N
NC, NS = 2, 16
NW = NC * NS
CHUNK = 128
ROWS_PER_TILE = N_PAD // NS
BLK = 1024


def _fill_rows(aux, row, buf):
    def body(r, carry):
        pltpu.sync_copy(aux.at[row], buf.at[r])
        return carry

    lax.fori_loop(0, buf.shape[0], body, 0)


def _degree_body(src3, dst3, hs_out, hd_out, sidx, didx, hist_s, hist_d):
    c = lax.axis_index("c")
    s = lax.axis_index("s")
    wid = c * NS + s
    nchunks = sidx.shape[0]
    pltpu.sync_copy(src3.at[wid], sidx)
    pltpu.sync_copy(dst3.at[wid], didx)

    def zbody(i, carry):
        hist_s[pl.ds(i * 16, 16)] = jnp.zeros((16,), jnp.float32)
        hist_d[pl.ds(i * 16, 16)] = jnp.zeros((16,), jnp.float32)
        return carry

    lax.fori_loop(0, N_PAD // 16, zbody, 0)

    def body(j, carry):
        for k in range(CHUNK // 16):
            v = sidx[j, pl.ds(k * 16, 16)]
            cnt, last = plsc.scan_count(v)
            plsc.addupdate_scatter(hist_s, [v], cnt.astype(jnp.float32),
                                   mask=last)
            w = didx[j, pl.ds(k * 16, 16)]
            cnt2, last2 = plsc.scan_count(w)
            plsc.addupdate_scatter(hist_d, [w], cnt2.astype(jnp.float32),
                                   mask=last2)
        return carry

    lax.fori_loop(0, nchunks, body, 0)
    pltpu.sync_copy(hist_s, hs_out.at[wid])
    pltpu.sync_copy(hist_d, hd_out.at[wid])


PH = 16


def _gather2(h, sidx, j, rows, sem):
    half = CHUNK // 2
    pltpu.async_copy(h.at[sidx.at[j, pl.ds(0, half)]],
                     rows.at[pl.ds(0, half)], sem)
    pltpu.async_copy(h.at[sidx.at[j, pl.ds(half, half)]],
                     rows.at[pl.ds(half, half)], sem)


def _agg_body(h, src3, dst3, aux, aggp,
              sidx, didx, rows0, rows1, agg_sh, gsem0, gsem1, asem0, asem1):
    c = lax.axis_index("c")
    s = lax.axis_index("s")
    wid = c * NS + s
    nchunks = src3.shape[1]
    r0 = s * ROWS_PER_TILE
    _fill_rows(aux, 0, rows0)
    for t in range(ROWS_PER_TILE // CHUNK):
        pltpu.async_copy(rows0, agg_sh.at[pl.ds(r0 + t * CHUNK, CHUNK)],
                         gsem0)
    for t in range(ROWS_PER_TILE // CHUNK):
        pltpu.make_async_copy(rows0,
                              agg_sh.at[pl.ds(r0 + t * CHUNK, CHUNK)],
                              gsem0).wait()
    plsc.subcore_barrier()

    def phase_body(ph, carry):
        base = ph * PH
        pltpu.sync_copy(src3.at[wid].at[pl.ds(base, PH)], sidx)
        pltpu.sync_copy(dst3.at[wid].at[pl.ds(base, PH)], didx)
        _gather2(h, sidx, 0, rows0, gsem0)
        _gather2(h, sidx, 1, rows1, gsem1)

        def body(p, c2):
            j0 = 2 * p
            j1 = j0 + 1
            pltpu.make_async_copy(h.at[sidx.at[j0]], rows0, gsem0).wait()
            pltpu.async_copy(rows0, agg_sh.at[didx.at[j0]], asem0, add=True)
            pltpu.make_async_copy(h.at[sidx.at[j1]], rows1, gsem1).wait()
            pltpu.async_copy(rows1, agg_sh.at[didx.at[j1]], asem1, add=True)
            pltpu.make_async_copy(rows0, agg_sh.at[didx.at[j0]], asem0).wait()
            pltpu.make_async_copy(rows1, agg_sh.at[didx.at[j1]], asem1).wait()

            @pl.when(p + 1 < PH // 2)
            def _():
                _gather2(h, sidx, j0 + 2, rows0, gsem0)
                _gather2(h, sidx, j1 + 2, rows1, gsem1)

            return c2

        lax.fori_loop(0, PH // 2, body, 0)
        return carry

    lax.fori_loop(0, nchunks // PH, phase_body, 0)
    plsc.subcore_barrier()
    pltpu.sync_copy(agg_sh.at[pl.ds(r0, ROWS_PER_TILE)],
                    aggp.at[c].at[pl.ds(r0, ROWS_PER_TILE)])


def _mm_norm_body(hs_ref, x_ref, w_ref, o_ref):
    degs = jnp.sum(hs_ref[...], axis=0)
    norm = lax.rsqrt(jnp.maximum(degs, 1.0))
    o_ref[...] = jnp.dot(x_ref[...], w_ref[...],
                         preferred_element_type=jnp.float32) * norm[:, None]


def _mid_body(aggp_ref, hs_ref, hd_ref, b_ref, w_ref, o_ref):
    nd = lax.rsqrt(jnp.maximum(jnp.sum(hd_ref[...], axis=0), 1.0))
    ns = lax.rsqrt(jnp.maximum(jnp.sum(hs_ref[...], axis=0), 1.0))
    agg = aggp_ref[0] + aggp_ref[1]
    g = jnp.maximum(agg * nd[:, None] + b_ref[...], 0.0)
    o_ref[...] = jnp.dot(g, w_ref[...],
                         preferred_element_type=jnp.float32) * ns[:, None]


def _fin_body(aggp_ref, hd_ref, b_ref, o_ref):
    nd = lax.rsqrt(jnp.maximum(jnp.sum(hd_ref[...], axis=0), 1.0))
    o_ref[...] = (aggp_ref[0] + aggp_ref[1]) * nd[:, None] + b_ref[...]


def kernel(x, edge_index, W1, b1, W2, b2):
    mesh = plsc.VectorSubcoreMesh(core_axis_name="c", subcore_axis_name="s")
    src = edge_index[0].astype(jnp.int32)
    dst = edge_index[1].astype(jnp.int32)
    e0 = src.shape[0]
    ch = -(-e0 // (NW * CHUNK))
    ch = -(-ch // PH) * PH
    e_pad = NW * ch * CHUNK
    pad = TRASH + (jnp.arange(e_pad - e0, dtype=jnp.int32) % (N_PAD - N))
    src3 = jnp.concatenate([src, pad]).reshape(NW, ch, CHUNK)
    dst3 = jnp.concatenate([dst, pad]).reshape(NW, ch, CHUNK)
    x_pad = jnp.zeros((N_PAD, D), jnp.float32).at[:N].set(x)
    aux = jnp.zeros((4, D), jnp.float32).at[1, 0].set(1.0).at[2, 1].set(1.0)

    deg_call = pl.kernel(
        _degree_body,
        out_type=(jax.ShapeDtypeStruct((NW, N_PAD), jnp.float32),
                  jax.ShapeDtypeStruct((NW, N_PAD), jnp.float32)),
        mesh=mesh,
        scratch_types=[pltpu.VMEM((ch, CHUNK), jnp.int32),
                       pltpu.VMEM((ch, CHUNK), jnp.int32),
                       pltpu.VMEM((N_PAD,), jnp.float32),
                       pltpu.VMEM((N_PAD,), jnp.float32)],
        compiler_params=pltpu.CompilerParams(needs_layout_passes=False),
    )
    agg_call = pl.kernel(
        _agg_body,
        out_type=jax.ShapeDtypeStruct((NC, N_PAD, D), jnp.float32),
        mesh=mesh,
        scratch_types=[pltpu.VMEM((PH, CHUNK), jnp.int32),
                       pltpu.VMEM((PH, CHUNK), jnp.int32),
                       pltpu.VMEM((CHUNK, D), jnp.float32),
                       pltpu.VMEM((CHUNK, D), jnp.float32),
                       pltpu.VMEM_SHARED((N_PAD, D), jnp.float32),
                       pltpu.SemaphoreType.DMA,
                       pltpu.SemaphoreType.DMA,
                       pltpu.SemaphoreType.DMA,
                       pltpu.SemaphoreType.DMA],
    )

    hs_p, hd_p = deg_call(src3, dst3)

    grid = (N_PAD // BLK,)
    hist_spec = pl.BlockSpec((NW, BLK), lambda i: (0, i))
    row_spec = pl.BlockSpec((BLK, D), lambda i: (i, 0))
    w_spec = pl.BlockSpec((D, D), lambda i: (0, 0))
    b_spec = pl.BlockSpec((1, D), lambda i: (0, 0))
    agg_spec = pl.BlockSpec((2, BLK, D), lambda i: (0, i, 0))
    out_sds = jax.ShapeDtypeStruct((N_PAD, D), jnp.float32)

    h1 = pl.pallas_call(
        _mm_norm_body, grid=grid,
        in_specs=[hist_spec, row_spec, w_spec],
        out_specs=row_spec, out_shape=out_sds,
    )(hs_p, x_pad, W1)

    agg1 = agg_call(h1, src3, dst3, aux)

    h2 = pl.pallas_call(
        _mid_body, grid=grid,
        in_specs=[agg_spec, hist_spec, hist_spec, b_spec, w_spec],
        out_specs=row_spec, out_shape=out_sds,
    )(agg1, hs_p, hd_p, b1.reshape(1, D), W2)

    agg2 = agg_call(h2, src3, dst3, aux)

    out = pl.pallas_call(
        _fin_body, grid=grid,
        in_specs=[agg_spec, hist_spec, b_spec],
        out_specs=row_spec, out_shape=out_sds,
    )(agg2, hd_p, b2.reshape(1, D))

    return out[:N]


# speedup vs baseline: 1.0015x; 1.0015x over previous
import jax
import jax.numpy as jnp
from jax import lax
from jax.experimental import pallas as pl
from jax.experimental.pallas import tpu as pltpu
from jax.experimental.pallas import tpu_sc as plsc

N = 10000
D = 128
N_PAD = 10240
TRASH = N
NC, NS = 2, 16
NW = NC * NS
CHUNK = 128
ROWS_PER_TILE = N_PAD // NS
BLK = 1024


def _fill_rows(aux, buf):
    def body(r, carry):
        pltpu.sync_copy(aux.at[0], buf.at[r])
        return carry

    lax.fori_loop(0, buf.shape[0], body, 0)


def _degree_body(src3, dst3, hs_out, hd_out, sidx, didx, hist_s, hist_d):
    c = lax.axis_index("c")
    s = lax.axis_index("s")
    wid = c * NS + s
    nchunks = sidx.shape[0]
    pltpu.sync_copy(src3.at[wid], sidx)
    pltpu.sync_copy(dst3.at[wid], didx)

    def zbody(i, carry):
        hist_s[pl.ds(i * 16, 16)] = jnp.zeros((16,), jnp.float32)
        hist_d[pl.ds(i * 16, 16)] = jnp.zeros((16,), jnp.float32)
        return carry

    lax.fori_loop(0, N_PAD // 16, zbody, 0)

    def body(j, carry):
        for k in range(CHUNK // 16):
            v = sidx[j, pl.ds(k * 16, 16)]
            cnt, last = plsc.scan_count(v)
            plsc.addupdate_scatter(hist_s, [v], cnt.astype(jnp.float32),
                                   mask=last)
            w = didx[j, pl.ds(k * 16, 16)]
            cnt2, last2 = plsc.scan_count(w)
            plsc.addupdate_scatter(hist_d, [w], cnt2.astype(jnp.float32),
                                   mask=last2)
        return carry

    lax.fori_loop(0, nchunks, body, 0)
    pltpu.sync_copy(hist_s, hs_out.at[wid])
    pltpu.sync_copy(hist_d, hd_out.at[wid])


PH = 16


def _agg_body(h, src3, dst3, aux, aggp,
              sidx, didx, rows0, rows1, agg_sh, gsem0, gsem1, asem0, asem1):
    c = lax.axis_index("c")
    s = lax.axis_index("s")
    wid = c * NS + s
    nchunks = src3.shape[1]
    r0 = s * ROWS_PER_TILE
    _fill_rows(aux, rows0)
    for t in range(ROWS_PER_TILE // CHUNK):
        pltpu.async_copy(rows0, agg_sh.at[pl.ds(r0 + t * CHUNK, CHUNK)],
                         gsem0)
    for t in range(ROWS_PER_TILE // CHUNK):
        pltpu.make_async_copy(rows0,
                              agg_sh.at[pl.ds(r0 + t * CHUNK, CHUNK)],
                              gsem0).wait()
    plsc.subcore_barrier()

    def phase_body(ph, carry):
        base = ph * PH
        pltpu.sync_copy(src3.at[wid].at[pl.ds(base, PH)], sidx)
        pltpu.sync_copy(dst3.at[wid].at[pl.ds(base, PH)], didx)
        pltpu.async_copy(h.at[sidx.at[0]], rows0, gsem0)
        pltpu.async_copy(h.at[sidx.at[1]], rows1, gsem1)

        def body(p, c2):
            j0 = 2 * p
            j1 = j0 + 1
            pltpu.make_async_copy(h.at[sidx.at[j0]], rows0, gsem0).wait()
            pltpu.async_copy(rows0, agg_sh.at[didx.at[j0]], asem0, add=True)
            pltpu.make_async_copy(h.at[sidx.at[j1]], rows1, gsem1).wait()
            pltpu.async_copy(rows1, agg_sh.at[didx.at[j1]], asem1, add=True)
            pltpu.make_async_copy(rows0, agg_sh.at[didx.at[j0]], asem0).wait()
            pltpu.make_async_copy(rows1, agg_sh.at[didx.at[j1]], asem1).wait()

            @pl.when(p + 1 < PH // 2)
            def _():
                pltpu.async_copy(h.at[sidx.at[j0 + 2]], rows0, gsem0)
                pltpu.async_copy(h.at[sidx.at[j1 + 2]], rows1, gsem1)

            return c2

        lax.fori_loop(0, PH // 2, body, 0)
        return carry

    lax.fori_loop(0, nchunks // PH, phase_body, 0)
    plsc.subcore_barrier()
    pltpu.sync_copy(agg_sh.at[pl.ds(r0, ROWS_PER_TILE)],
                    aggp.at[c].at[pl.ds(r0, ROWS_PER_TILE)])


def _mm_norm_body(hs_ref, x_ref, w_ref, o_ref):
    degs = jnp.sum(hs_ref[...], axis=0)
    norm = lax.rsqrt(jnp.maximum(degs, 1.0))
    o_ref[...] = jnp.dot(x_ref[...], w_ref[...],
                         preferred_element_type=jnp.float32) * norm[:, None]


def _mid_body(aggp_ref, hs_ref, hd_ref, b_ref, w_ref, o_ref):
    nd = lax.rsqrt(jnp.maximum(jnp.sum(hd_ref[...], axis=0), 1.0))
    ns = lax.rsqrt(jnp.maximum(jnp.sum(hs_ref[...], axis=0), 1.0))
    agg = aggp_ref[0] + aggp_ref[1]
    g = jnp.maximum(agg * nd[:, None] + b_ref[...], 0.0)
    o_ref[...] = jnp.dot(g, w_ref[...],
                         preferred_element_type=jnp.float32) * ns[:, None]


def _fin_body(aggp_ref, hd_ref, b_ref, o_ref):
    nd = lax.rsqrt(jnp.maximum(jnp.sum(hd_ref[...], axis=0), 1.0))
    o_ref[...] = (aggp_ref[0] + aggp_ref[1]) * nd[:, None] + b_ref[...]


def kernel(x, edge_index, W1, b1, W2, b2):
    mesh = plsc.VectorSubcoreMesh(core_axis_name="c", subcore_axis_name="s")
    src = edge_index[0].astype(jnp.int32)
    dst = edge_index[1].astype(jnp.int32)
    e0 = src.shape[0]
    ch = -(-e0 // (NW * CHUNK))
    ch = -(-ch // PH) * PH
    e_pad = NW * ch * CHUNK
    pad = TRASH + (jnp.arange(e_pad - e0, dtype=jnp.int32) % (N_PAD - N))
    src3 = jnp.concatenate([src, pad]).reshape(NW, ch, CHUNK)
    dst3 = jnp.concatenate([dst, pad]).reshape(NW, ch, CHUNK)
    x_pad = jnp.zeros((N_PAD, D), jnp.float32).at[:N].set(x)
    aux = jnp.zeros((1, D), jnp.float32)

    deg_call = pl.kernel(
        _degree_body,
        out_type=(jax.ShapeDtypeStruct((NW, N_PAD), jnp.float32),
                  jax.ShapeDtypeStruct((NW, N_PAD), jnp.float32)),
        mesh=mesh,
        scratch_types=[pltpu.VMEM((ch, CHUNK), jnp.int32),
                       pltpu.VMEM((ch, CHUNK), jnp.int32),
                       pltpu.VMEM((N_PAD,), jnp.float32),
                       pltpu.VMEM((N_PAD,), jnp.float32)],
        compiler_params=pltpu.CompilerParams(needs_layout_passes=False),
    )
    agg_call = pl.kernel(
        _agg_body,
        out_type=jax.ShapeDtypeStruct((NC, N_PAD, D), jnp.float32),
        mesh=mesh,
        scratch_types=[pltpu.VMEM((PH, CHUNK), jnp.int32),
                       pltpu.VMEM((PH, CHUNK), jnp.int32),
                       pltpu.VMEM((CHUNK, D), jnp.float32),
                       pltpu.VMEM((CHUNK, D), jnp.float32),
                       pltpu.VMEM_SHARED((N_PAD, D), jnp.float32),
                       pltpu.SemaphoreType.DMA,
                       pltpu.SemaphoreType.DMA,
                       pltpu.SemaphoreType.DMA,
                       pltpu.SemaphoreType.DMA],
    )

    hs_p, hd_p = deg_call(src3, dst3)

    grid = (N_PAD // BLK,)
    hist_spec = pl.BlockSpec((NW, BLK), lambda i: (0, i))
    row_spec = pl.BlockSpec((BLK, D), lambda i: (i, 0))
    w_spec = pl.BlockSpec((D, D), lambda i: (0, 0))
    b_spec = pl.BlockSpec((1, D), lambda i: (0, 0))
    agg_spec = pl.BlockSpec((2, BLK, D), lambda i: (0, i, 0))
    out_sds = jax.ShapeDtypeStruct((N_PAD, D), jnp.float32)

    h1 = pl.pallas_call(
        _mm_norm_body, grid=grid,
        in_specs=[hist_spec, row_spec, w_spec],
        out_specs=row_spec, out_shape=out_sds,
    )(hs_p, x_pad, W1)

    agg1 = agg_call(h1, src3, dst3, aux)

    h2 = pl.pallas_call(
        _mid_body, grid=grid,
        in_specs=[agg_spec, hist_spec, hist_spec, b_spec, w_spec],
        out_specs=row_spec, out_shape=out_sds,
    )(agg1, hs_p, hd_p, b1.reshape(1, D), W2)

    agg2 = agg_call(h2, src3, dst3, aux)

    out = pl.pallas_call(
        _fin_body, grid=grid,
        in_specs=[agg_spec, hist_spec, b_spec],
        out_specs=row_spec, out_shape=out_sds,
    )(agg2, hd_p, b2.reshape(1, D))

    return out[:N]
